# baseline (speedup 1.0000x reference)
import functools

import jax
import jax.numpy as jnp
from jax import lax
from jax.experimental import pallas as pl
from jax.experimental.pallas import tpu as pltpu
from jax.experimental.pallas import tpu_sc as plsc

BATCH = 4096
SEQ = 50
DIM = 64
LANES = 16

NUM_CORES = 2
NUM_SUBCORES = 16
NW = NUM_CORES * NUM_SUBCORES
BBLK = BATCH // NW
NBUF = 5

_mesh = plsc.VectorSubcoreMesh(core_axis_name="c", subcore_axis_name="s")


@functools.partial(
    pl.kernel,
    out_type=jax.ShapeDtypeStruct((SEQ, DIM, BATCH), jnp.float32),
    mesh=_mesh,
    compiler_params=pltpu.CompilerParams(
        use_tc_tiling_on_sc=False, needs_layout_passes=False
    ),
    scratch_types=[
        pltpu.VMEM((SEQ, BBLK), jnp.int32),
        [pltpu.VMEM((BBLK, DIM), jnp.float32) for _ in range(NBUF)],
        [pltpu.VMEM((DIM, BBLK), jnp.float32) for _ in range(NBUF)],
        [pltpu.SemaphoreType.DMA for _ in range(NBUF)],
        [pltpu.SemaphoreType.DMA for _ in range(NBUF)],
    ],
)
def _gather_rows(idx_hbm, table_hbm, out_hbm, idx_v, bufs, obufs, gsems, wsems):
    wid = lax.axis_index("s") * NUM_CORES + lax.axis_index("c")
    wb = wid * BBLK
    pltpu.sync_copy(idx_hbm.at[:, pl.ds(wb, BBLK)], idx_v)

    lane = lax.broadcasted_iota(jnp.int32, (LANES,), 0)

    def start_gather(s, b):
        pltpu.async_copy(table_hbm.at[idx_v.at[s]], bufs[b], gsems[b])

    def drain_gather(b):
        pltpu.make_async_copy(table_hbm.at[pl.ds(0, BBLK)], bufs[b], gsems[b]).wait()

    def drain_write(b):
        pltpu.make_async_copy(
            out_hbm.at[0, :, pl.ds(0, BBLK)], obufs[b], wsems[b]
        ).wait()

    def finish(s, b, first, do_gather):
        drain_gather(b)
        if not first:
            drain_write(b)

        @pl.loop(0, DIM, unroll=4)
        def _transpose(d):
            col = jnp.full((LANES,), d, dtype=jnp.int32)
            for j in range(BBLK // LANES):
                row = lane + (j * LANES)
                vals = plsc.load_gather(bufs[b], [row, col])
                obufs[b][d, pl.ds(j * LANES, LANES)] = vals

        pltpu.async_copy(obufs[b], out_hbm.at[s, :, pl.ds(wb, BBLK)], wsems[b])
        if do_gather:
            start_gather(s + NBUF, b)

    for b in range(NBUF):
        start_gather(b, b)

    for b in range(NBUF):
        finish(b, b, True, True)

    @pl.loop(0, (SEQ - NBUF) // NBUF - 1)
    def _main(g):
        for b in range(NBUF):
            finish(NBUF + g * NBUF + b, b, False, True)

    for b in range(NBUF):
        finish(SEQ - NBUF + b, b, False, False)

    for b in range(NBUF):
        drain_write(b)


def kernel(IOs, table):
    y = _gather_rows(IOs.T.astype(jnp.int32), table)
    return y.transpose(2, 0, 1)


# speedup vs baseline: 1.4981x; 1.4981x over previous
import functools

import jax
import jax.numpy as jnp
from jax import lax
from jax.experimental import pallas as pl
from jax.experimental.pallas import tpu as pltpu
from jax.experimental.pallas import tpu_sc as plsc

BATCH = 4096
SEQ = 50
DIM = 64
LANES = 16

NUM_CORES = 2
NUM_SUBCORES = 16
NW = NUM_CORES * NUM_SUBCORES
BBLK = BATCH // NW
NBUF = 5

_mesh = plsc.VectorSubcoreMesh(core_axis_name="c", subcore_axis_name="s")


@functools.partial(
    pl.kernel,
    out_type=jax.ShapeDtypeStruct((SEQ, DIM, BATCH), jnp.float32),
    mesh=_mesh,
    compiler_params=pltpu.CompilerParams(
        use_tc_tiling_on_sc=False, needs_layout_passes=False
    ),
    scratch_types=[
        pltpu.VMEM((SEQ, BBLK), jnp.int32),
        [pltpu.VMEM((BBLK, DIM), jnp.float32) for _ in range(NBUF)],
        [pltpu.VMEM((DIM, BBLK), jnp.float32) for _ in range(NBUF)],
        [pltpu.SemaphoreType.DMA for _ in range(NBUF)],
        [pltpu.SemaphoreType.DMA for _ in range(NBUF)],
    ],
)
def _gather_rows(idx_hbm, table_hbm, out_hbm, idx_v, bufs, obufs, gsems, wsems):
    wid = lax.axis_index("s") * NUM_CORES + lax.axis_index("c")
    wb = wid * BBLK
    pltpu.sync_copy(idx_hbm.at[:, pl.ds(wb, BBLK)], idx_v)

    lane = lax.broadcasted_iota(jnp.int32, (LANES,), 0)

    def start_gather(s, b):
        pltpu.async_copy(table_hbm.at[idx_v.at[s]], bufs[b], gsems[b])

    def drain_gather(b):
        pltpu.make_async_copy(table_hbm.at[pl.ds(0, BBLK)], bufs[b], gsems[b]).wait()

    def drain_write(b):
        pltpu.make_async_copy(
            out_hbm.at[0, :, pl.ds(0, BBLK)], obufs[b], wsems[b]
        ).wait()

    def finish(s, b, first, do_gather):
        drain_gather(b)
        if not first:
            drain_write(b)

        @plsc.parallel_loop(0, DIM, unroll=4)
        def _transpose(d):
            col = jnp.full((LANES,), d, dtype=jnp.int32)
            for j in range(BBLK // LANES):
                row = lane + (j * LANES)
                vals = plsc.load_gather(bufs[b], [row, col])
                obufs[b][d, pl.ds(j * LANES, LANES)] = vals

        pltpu.async_copy(obufs[b], out_hbm.at[s, :, pl.ds(wb, BBLK)], wsems[b])
        if do_gather:
            start_gather(s + NBUF, b)

    for b in range(NBUF):
        start_gather(b, b)

    for b in range(NBUF):
        finish(b, b, True, True)

    @pl.loop(0, (SEQ - NBUF) // NBUF - 1)
    def _main(g):
        for b in range(NBUF):
            finish(NBUF + g * NBUF + b, b, False, True)

    for b in range(NBUF):
        finish(SEQ - NBUF + b, b, False, False)

    for b in range(NBUF):
        drain_write(b)


def kernel(IOs, table):
    y = _gather_rows(IOs.T.astype(jnp.int32), table)
    return y.transpose(2, 0, 1)
